# baseline (speedup 1.0000x reference)
import functools

import jax
import jax.numpy as jnp
from jax import lax
from jax.experimental import pallas as pl
from jax.experimental.pallas import tpu as pltpu
from jax.experimental.pallas import tpu_sc as plsc

B = 16384
D = 32
NC = 2
NS = 16
NW = NC * NS
BPW = B // NW
CH = 128
NCHUNK = BPW // CH
NBLK = BPW // 16

_mesh = plsc.VectorSubcoreMesh(core_axis_name="c", subcore_axis_name="s")


@functools.partial(
    pl.kernel,
    mesh=_mesh,
    compiler_params=pltpu.CompilerParams(
        needs_layout_passes=False, use_tc_tiling_on_sc=False),
    out_type=jax.ShapeDtypeStruct((B,), jnp.float32),
    scratch_types=[
        pltpu.VMEM((NCHUNK, CH), jnp.int32),
        pltpu.VMEM((NCHUNK, CH), jnp.int32),
        pltpu.VMEM((BPW, D), jnp.float32),
        pltpu.VMEM((BPW, D), jnp.float32),
        pltpu.VMEM((BPW,), jnp.float32),
        pltpu.VMEM((16,), jnp.float32),
        pltpu.VMEM((16,), jnp.float32),
        pltpu.SemaphoreType.DMA,
        pltpu.SemaphoreType.DMA,
    ],
)
def _sc_kernel(uids_hbm, mids_hbm, utab_hbm, mtab_hbm, wv_hbm, bv_hbm,
               out_hbm, uidx_v, midx_v, urows_v, mrows_v, out_v,
               wv_v, bv_v, sem_u, sem_m):
    wid = lax.axis_index("s") * NC + lax.axis_index("c")
    cbase = wid * NCHUNK

    pltpu.sync_copy(uids_hbm.at[pl.ds(cbase, NCHUNK)], uidx_v)
    pltpu.sync_copy(mids_hbm.at[pl.ds(cbase, NCHUNK)], midx_v)
    pltpu.sync_copy(wv_hbm, wv_v)
    pltpu.sync_copy(bv_hbm, bv_v)

    copies = []
    for j in range(NCHUNK):
        copies.append(pltpu.async_copy(
            utab_hbm.at[uidx_v.at[j]], urows_v.at[pl.ds(j * CH, CH)], sem_u))
        copies.append(pltpu.async_copy(
            mtab_hbm.at[midx_v.at[j]], mrows_v.at[pl.ds(j * CH, CH)], sem_m))
    for c in copies:
        c.wait()

    wv = wv_v[...]
    bv = bv_v[...]
    lanes = lax.iota(jnp.int32, 16)

    def blk_body(i, carry):
        rows = i * 16 + lanes
        acc = jnp.zeros((16,), jnp.float32)
        for col in range(D):
            cols = jnp.full((16,), col, jnp.int32)
            uv = plsc.load_gather(urows_v, [rows, cols])
            mv = plsc.load_gather(mrows_v, [rows, cols])
            acc = acc + uv * mv
        z = acc * wv + bv
        out_v[pl.ds(i * 16, 16)] = 1.0 / (1.0 + jnp.exp(-z))
        return carry

    lax.fori_loop(0, NBLK, blk_body, 0)
    pltpu.sync_copy(out_v, out_hbm.at[pl.ds(wid * BPW, BPW)])


def kernel(x, user_table, movie_table, W, b):
    xi = x.astype(jnp.int32)
    uids = xi[0].reshape(NW * NCHUNK, CH)
    mids = xi[1].reshape(NW * NCHUNK, CH)
    wv = jnp.broadcast_to(W.reshape(-1)[0], (16,)).astype(jnp.float32)
    bv = jnp.broadcast_to(b.reshape(-1)[0], (16,)).astype(jnp.float32)
    out = _sc_kernel(uids, mids, user_table.astype(jnp.float32),
                     movie_table.astype(jnp.float32), wv, bv)
    return out.reshape(B, 1)
